# Initial kernel scaffold; baseline (speedup 1.0000x reference)
import jax
import jax.numpy as jnp
from jax.experimental import pallas as pl


def kernel(position_ids, table):
    raise NotImplementedError("write your pallas kernel here")


# speedup vs baseline: 2.3726x; 2.3726x over previous
import jax
import jax.numpy as jnp
from jax import lax
from jax.experimental import pallas as pl
from jax.experimental.pallas import tpu as pltpu
from jax.experimental.pallas import tpu_sc as plsc

EMBED_DIM = 1024
NC = 2
NS = 16
NW = NC * NS

CHUNK = 32
NBUF = 2


def _make_gather(b_total):
    b_per_w = b_total // NW
    nchunk = b_per_w // CHUNK
    niter = nchunk // NBUF

    mesh = plsc.VectorSubcoreMesh(core_axis_name="c", subcore_axis_name="s")

    def body(table_hbm, idx_hbm, out_hbm, idx_v, rows_v, *sems):
        sem_in = sems[:NBUF]
        sem_out = sems[NBUF:]
        wid = lax.axis_index("s") * NC + lax.axis_index("c")
        base = wid * b_per_w

        pltpu.sync_copy(idx_hbm.at[wid], idx_v)

        def start_in(g, b):
            pltpu.async_copy(table_hbm.at[idx_v.at[g]], rows_v.at[b], sem_in[b])

        def wait_in(g, b):
            pltpu.make_async_copy(
                table_hbm.at[idx_v.at[g]], rows_v.at[b], sem_in[b]).wait()

        def start_out(g, b):
            pltpu.async_copy(
                rows_v.at[b], out_hbm.at[pl.ds(base + g * CHUNK, CHUNK)],
                sem_out[b])

        def wait_out(g, b):
            pltpu.make_async_copy(
                rows_v.at[b], out_hbm.at[pl.ds(base + g * CHUNK, CHUNK)],
                sem_out[b]).wait()

        for b in range(NBUF):
            start_in(b, b)

        def step(t, carry):
            for b in range(NBUF):
                g = t * NBUF + b
                wait_in(g, b)
                start_out(g, b)
                wait_out(g, b)
                start_in(g + NBUF, b)
            return carry

        lax.fori_loop(0, niter - 1, step, 0, unroll=False)

        for b in range(NBUF):
            g = nchunk - NBUF + b
            wait_in(g, b)
            start_out(g, b)
            wait_out(g, b)

    scratch = [
        pltpu.VMEM((nchunk, CHUNK), jnp.int32),
        pltpu.VMEM((NBUF, CHUNK, EMBED_DIM), jnp.float32),
    ] + [pltpu.SemaphoreType.DMA] * (2 * NBUF)

    return pl.kernel(
        body,
        out_type=jax.ShapeDtypeStruct((b_total, EMBED_DIM), jnp.float32),
        mesh=mesh,
        scratch_types=scratch,
    )


@jax.jit
def kernel(position_ids, table):
    b_total = position_ids.size
    idx = position_ids.reshape(NW, (b_total // NW) // CHUNK, CHUNK)
    idx = idx.astype(jnp.int32)
    out = _make_gather(b_total)(table, idx)
    return out.reshape(position_ids.shape + (EMBED_DIM,))
